# baseline (speedup 1.0000x reference)
import jax
import jax.numpy as jnp
from jax.experimental import pallas as pl
from jax.experimental.pallas import tpu as pltpu

_PACK = 32
_K1 = _PACK * 12
_N1 = _PACK * 128
_TILE_P = 256


def _mlp_kernel(xp_ref, w1p_ref, b1t_ref, w2p_ref, o_ref):
    xp = xp_ref[...].astype(jnp.bfloat16)
    h = jnp.dot(xp, w1p_ref[...], preferred_element_type=jnp.float32)
    h = jnp.maximum(h + b1t_ref[...], 0.0).astype(jnp.bfloat16)
    o_ref[...] = jnp.dot(h, w2p_ref[...], preferred_element_type=jnp.float32)


def kernel(x, w1_aug, w2_aug):
    x = jnp.asarray(x, jnp.float32)
    B = x.shape[0]

    rows_per_step = _PACK * _TILE_P
    B_pad = ((B + rows_per_step - 1) // rows_per_step) * rows_per_step
    if B_pad != B:
        x = jnp.pad(x, ((0, B_pad - B), (0, 0)))
    P = B_pad // _PACK

    xp = x.reshape(P, _K1)

    eye = jnp.eye(_PACK, dtype=jnp.float32)
    w1p = jnp.kron(eye, w1_aug[:12, :]).astype(jnp.bfloat16)
    b1t = jnp.tile(w1_aug[12:13, :], (1, _PACK))
    w2p = jnp.kron(eye, w2_aug[:, :4]).astype(jnp.bfloat16)

    out = pl.pallas_call(
        _mlp_kernel,
        out_shape=jax.ShapeDtypeStruct((P, 128), jnp.float32),
        grid=(P // _TILE_P,),
        in_specs=[
            pl.BlockSpec((_TILE_P, _K1), lambda i: (i, 0)),
            pl.BlockSpec((_K1, _N1), lambda i: (0, 0)),
            pl.BlockSpec((1, _N1), lambda i: (0, 0)),
            pl.BlockSpec((_N1, 128), lambda i: (0, 0)),
        ],
        out_specs=pl.BlockSpec((_TILE_P, 128), lambda i: (i, 0)),
        compiler_params=pltpu.CompilerParams(
            dimension_semantics=("parallel",)
        ),
    )(xp, w1p, b1t, w2p)

    return out.reshape(B_pad, 4)[:B]


# speedup vs baseline: 3.9765x; 3.9765x over previous
import jax
import jax.numpy as jnp
from jax.experimental import pallas as pl
from jax.experimental.pallas import tpu as pltpu

_TILE = 2048


def _mlp_kernel(xt_ref, w1t_ref, b1c_ref, w2t_ref, o_ref):
    h = jax.lax.dot_general(
        w1t_ref[...], xt_ref[...], (((1,), (0,)), ((), ())),
        preferred_element_type=jnp.float32,
    )
    h = jnp.maximum(h + b1c_ref[...], 0.0)
    o_ref[...] = jax.lax.dot_general(
        w2t_ref[...], h, (((1,), (0,)), ((), ())),
        preferred_element_type=jnp.float32,
    )


def kernel(x, w1_aug, w2_aug):
    x = jnp.asarray(x, jnp.float32)
    B = x.shape[0]
    B_pad = ((B + _TILE - 1) // _TILE) * _TILE

    xt = x.T
    if B_pad != B:
        xt = jnp.pad(xt, ((0, 0), (0, B_pad - B)))

    w1t = w1_aug[:12, :].T
    b1c = w1_aug[12:13, :].T
    w2t = jnp.zeros((8, 128), jnp.float32).at[:4, :].set(w2_aug[:, :4].T)

    ot = pl.pallas_call(
        _mlp_kernel,
        out_shape=jax.ShapeDtypeStruct((8, B_pad), jnp.float32),
        grid=(B_pad // _TILE,),
        in_specs=[
            pl.BlockSpec((12, _TILE), lambda i: (0, i)),
            pl.BlockSpec((128, 12), lambda i: (0, 0)),
            pl.BlockSpec((128, 1), lambda i: (0, 0)),
            pl.BlockSpec((8, 128), lambda i: (0, 0)),
        ],
        out_specs=pl.BlockSpec((8, _TILE), lambda i: (0, i)),
        compiler_params=pltpu.CompilerParams(
            dimension_semantics=("parallel",)
        ),
    )(xt, w1t, b1c, w2t)

    return ot[:4, :B].T


# speedup vs baseline: 8.8612x; 2.2284x over previous
import jax
import jax.numpy as jnp
from jax.experimental import pallas as pl
from jax.experimental.pallas import tpu as pltpu

_TILE = 8192


def _mlp_kernel(xt_ref, w1t_ref, b1c_ref, w2t_ref, o_ref):
    h = jax.lax.dot_general(
        w1t_ref[...], xt_ref[...], (((1,), (0,)), ((), ())),
        preferred_element_type=jnp.float32,
    )
    h = jnp.maximum(h + b1c_ref[...], 0.0)
    o_ref[...] = jax.lax.dot_general(
        w2t_ref[...], h, (((1,), (0,)), ((), ())),
        preferred_element_type=jnp.float32,
    )


def kernel(x, w1_aug, w2_aug):
    x = jnp.asarray(x, jnp.float32)
    B = x.shape[0]
    B_pad = ((B + _TILE - 1) // _TILE) * _TILE

    xt = x.T
    if B_pad != B:
        xt = jnp.pad(xt, ((0, 0), (0, B_pad - B)))

    w1t = w1_aug[:12, :].T
    b1c = w1_aug[12:13, :].T
    w2t = jnp.zeros((8, 128), jnp.float32).at[:4, :].set(w2_aug[:, :4].T)

    ot = pl.pallas_call(
        _mlp_kernel,
        out_shape=jax.ShapeDtypeStruct((8, B_pad), jnp.float32),
        grid=(B_pad // _TILE,),
        in_specs=[
            pl.BlockSpec((12, _TILE), lambda i: (0, i)),
            pl.BlockSpec((128, 12), lambda i: (0, 0)),
            pl.BlockSpec((128, 1), lambda i: (0, 0)),
            pl.BlockSpec((8, 128), lambda i: (0, 0)),
        ],
        out_specs=pl.BlockSpec((8, _TILE), lambda i: (0, i)),
        compiler_params=pltpu.CompilerParams(
            dimension_semantics=("parallel",)
        ),
    )(xt, w1t, b1c, w2t)

    return ot[:4, :B].T


# speedup vs baseline: 10.3445x; 1.1674x over previous
import jax
import jax.numpy as jnp
from jax.experimental import pallas as pl
from jax.experimental.pallas import tpu as pltpu

_TILE = 16384


def _mlp_kernel(xt_ref, w1t_ref, b1c_ref, w2t_ref, o_ref):
    h = jax.lax.dot_general(
        w1t_ref[...], xt_ref[...], (((1,), (0,)), ((), ())),
        preferred_element_type=jnp.float32,
    )
    h = jnp.maximum(h + b1c_ref[...], 0.0)
    o_ref[...] = jax.lax.dot_general(
        w2t_ref[...], h, (((1,), (0,)), ((), ())),
        preferred_element_type=jnp.float32,
    )


def kernel(x, w1_aug, w2_aug):
    x = jnp.asarray(x, jnp.float32)
    B = x.shape[0]
    B_pad = ((B + _TILE - 1) // _TILE) * _TILE

    xt = x.T
    if B_pad != B:
        xt = jnp.pad(xt, ((0, 0), (0, B_pad - B)))

    w1t = w1_aug[:12, :].T
    b1c = w1_aug[12:13, :].T
    w2t = jnp.zeros((8, 128), jnp.float32).at[:4, :].set(w2_aug[:, :4].T)

    ot = pl.pallas_call(
        _mlp_kernel,
        out_shape=jax.ShapeDtypeStruct((8, B_pad), jnp.float32),
        grid=(B_pad // _TILE,),
        in_specs=[
            pl.BlockSpec((12, _TILE), lambda i: (0, i)),
            pl.BlockSpec((128, 12), lambda i: (0, 0)),
            pl.BlockSpec((128, 1), lambda i: (0, 0)),
            pl.BlockSpec((8, 128), lambda i: (0, 0)),
        ],
        out_specs=pl.BlockSpec((8, _TILE), lambda i: (0, i)),
        compiler_params=pltpu.CompilerParams(
            dimension_semantics=("parallel",)
        ),
    )(xt, w1t, b1c, w2t)

    return ot[:4, :B].T


# speedup vs baseline: 10.6422x; 1.0288x over previous
import jax
import jax.numpy as jnp
from jax.experimental import pallas as pl
from jax.experimental.pallas import tpu as pltpu

_TILE = 32768


def _mlp_kernel(xt_ref, w1t_ref, b1c_ref, w2t_ref, o_ref):
    h = jax.lax.dot_general(
        w1t_ref[...], xt_ref[...], (((1,), (0,)), ((), ())),
        preferred_element_type=jnp.float32,
    )
    h = jnp.maximum(h + b1c_ref[...], 0.0)
    o_ref[...] = jax.lax.dot_general(
        w2t_ref[...], h, (((1,), (0,)), ((), ())),
        preferred_element_type=jnp.float32,
    )


def kernel(x, w1_aug, w2_aug):
    x = jnp.asarray(x, jnp.float32)
    B = x.shape[0]
    B_pad = ((B + _TILE - 1) // _TILE) * _TILE

    xt = x.T
    if B_pad != B:
        xt = jnp.pad(xt, ((0, 0), (0, B_pad - B)))

    w1t = w1_aug[:12, :].T
    b1c = w1_aug[12:13, :].T
    w2t = jnp.zeros((8, 128), jnp.float32).at[:4, :].set(w2_aug[:, :4].T)

    ot = pl.pallas_call(
        _mlp_kernel,
        out_shape=jax.ShapeDtypeStruct((8, B_pad), jnp.float32),
        grid=(B_pad // _TILE,),
        in_specs=[
            pl.BlockSpec((12, _TILE), lambda i: (0, i)),
            pl.BlockSpec((128, 12), lambda i: (0, 0)),
            pl.BlockSpec((128, 1), lambda i: (0, 0)),
            pl.BlockSpec((8, 128), lambda i: (0, 0)),
        ],
        out_specs=pl.BlockSpec((8, _TILE), lambda i: (0, i)),
        compiler_params=pltpu.CompilerParams(
            dimension_semantics=("parallel",)
        ),
    )(xt, w1t, b1c, w2t)

    return ot[:4, :B].T
